# baseline (speedup 1.0000x reference)
import functools
import math

import jax
import jax.numpy as jnp
from jax import lax
from jax.experimental import pallas as pl
from jax.experimental.pallas import tpu as pltpu
from jax.experimental.pallas import tpu_sc as plsc

VOCAB = 100000
DIM = 64
BATCH = 16384
NUM_SAMPLED = 64

NC, NS = 2, 16
NW = NC * NS
BPW = BATCH // NW
CHUNK = 128
NCHUNK = BPW // CHUNK

_LOG_V1 = math.log(float(VOCAB) + 1.0)


VB = 16384
NVB = -(-VOCAB // VB)


def _prep_body(eT_ref, wT_ref, out_ref):
    out_ref[...] = jnp.concatenate([eT_ref[...].T, wT_ref[...].T], axis=1)


_prep = pl.pallas_call(
    _prep_body,
    grid=(NVB,),
    in_specs=[
        pl.BlockSpec((DIM, VB), lambda i: (0, i)),
        pl.BlockSpec((DIM, VB), lambda i: (0, i)),
    ],
    out_specs=pl.BlockSpec((VB, 128), lambda i: (i, 0)),
    out_shape=jax.ShapeDtypeStruct((VOCAB, 128), jnp.float32),
    compiler_params=pltpu.CompilerParams(vmem_limit_bytes=100 * 1024 * 1024),
)


def _sc_gather_body(inputs_h, labels_h, tab_h, ncb_h, sampled_h,
                    p_o, tb_o, s_o, sb_o,
                    idx_v, lidx_v, a_v, b_v, tb_v, sidx_v, s_v, sb_v,
                    sems, semt):
    wid = lax.axis_index("s") * NC + lax.axis_index("c")
    base = wid * BPW
    for j in range(NCHUNK):
        pltpu.sync_copy(inputs_h.at[pl.ds(base + j * CHUNK, CHUNK)], idx_v.at[j])
        pltpu.sync_copy(labels_h.at[pl.ds(base + j * CHUNK, CHUNK)], lidx_v.at[j])
    tb_handles = [
        pltpu.async_copy(ncb_h.at[lidx_v.at[j]],
                         tb_v.at[pl.ds(j * CHUNK, CHUNK)], semt)
        for j in range(NCHUNK)
    ]
    handles = [None] * NCHUNK

    def fire(j):
        s = j % 2
        handles[j] = (
            pltpu.async_copy(tab_h.at[idx_v.at[j]], a_v.at[s], sems.at[2 * s]),
            pltpu.async_copy(tab_h.at[lidx_v.at[j]], b_v.at[s], sems.at[2 * s + 1]),
        )

    def drain(j):
        ha, hb = handles[j]
        ha.wait()
        hb.wait()
        sl = pl.ds(base + j * CHUNK, CHUNK)
        s = j % 2
        pltpu.sync_copy(a_v.at[s, :, pl.ds(0, DIM)], p_o.at[sl, pl.ds(0, DIM)])
        pltpu.sync_copy(b_v.at[s, :, pl.ds(DIM, DIM)], p_o.at[sl, pl.ds(DIM, DIM)])

    fire(0)
    for j in range(1, NCHUNK):
        fire(j)
        drain(j - 1)
    drain(NCHUNK - 1)

    for h in tb_handles:
        h.wait()
    pltpu.sync_copy(tb_v, tb_o.at[pl.ds(base, BPW)])

    @pl.when(wid == 0)
    def _():
        pltpu.sync_copy(sampled_h, sidx_v)
        pltpu.async_copy(tab_h.at[sidx_v], s_v, semt).wait()
        pltpu.async_copy(ncb_h.at[sidx_v], sb_v, semt).wait()
        pltpu.sync_copy(s_v, s_o)
        pltpu.sync_copy(sb_v, sb_o)


@functools.cache
def _make_sc_gather():
    return functools.partial(
        pl.kernel,
        out_type=[
            jax.ShapeDtypeStruct((BATCH, 128), jnp.float32),
            jax.ShapeDtypeStruct((BATCH,), jnp.float32),
            jax.ShapeDtypeStruct((NUM_SAMPLED, 128), jnp.float32),
            jax.ShapeDtypeStruct((NUM_SAMPLED,), jnp.float32),
        ],
        mesh=plsc.VectorSubcoreMesh(core_axis_name="c", subcore_axis_name="s",
                                    num_cores=NC, num_subcores=NS),
        compiler_params=pltpu.CompilerParams(use_tc_tiling_on_sc=False),
        scratch_types=[
            pltpu.VMEM((NCHUNK, CHUNK), jnp.int32),
            pltpu.VMEM((NCHUNK, CHUNK), jnp.int32),
            pltpu.VMEM((2, CHUNK, 128), jnp.float32),
            pltpu.VMEM((2, CHUNK, 128), jnp.float32),
            pltpu.VMEM((BPW,), jnp.float32),
            pltpu.VMEM((NUM_SAMPLED,), jnp.int32),
            pltpu.VMEM((NUM_SAMPLED, 128), jnp.float32),
            pltpu.VMEM((NUM_SAMPLED,), jnp.float32),
            pltpu.SemaphoreType.DMA((4,)),
            pltpu.SemaphoreType.DMA,
        ],
    )(_sc_gather_body)


BLK = 2048
NBLK = BATCH // BLK


def _tc_loss_body(labels_ref, p_ref, tb_ref, s_ref, sbt_ref,
                  st_ref, out_ref):
    i = pl.program_id(0)
    emb = p_ref[:, pl.ds(0, DIM)]
    tw = p_ref[:, pl.ds(DIM, DIM)]
    tb = tb_ref[...]
    labf = labels_ref[...].astype(jnp.float32)

    true_q = (jnp.log(labf + 2.0) - jnp.log(labf + 1.0)) / _LOG_V1 * NUM_SAMPLED
    tdot = jnp.sum(emb * tw, axis=1, keepdims=True).T
    tl = tdot + tb - jnp.log(true_q)
    ce_true = jnp.maximum(tl, 0.0) - tl + jnp.log1p(jnp.exp(-jnp.abs(tl)))

    sf = st_ref[...].astype(jnp.float32)
    sq = (jnp.log(sf + 2.0) - jnp.log(sf + 1.0)) / _LOG_V1 * NUM_SAMPLED
    sw = s_ref[:, pl.ds(DIM, DIM)]
    slog = lax.dot_general(emb, sw, (((1,), (1,)), ((), ())),
                           preferred_element_type=jnp.float32)
    sl = slog + sbt_ref[...] - jnp.log(sq)
    ce_samp = jnp.maximum(sl, 0.0) + jnp.log1p(jnp.exp(-jnp.abs(sl)))

    part = jnp.broadcast_to(jnp.sum(ce_samp) + jnp.sum(ce_true), (1, 1))

    @pl.when(i == 0)
    def _():
        out_ref[...] = jnp.zeros((1, 1), jnp.float32)

    out_ref[...] += part

    @pl.when(i == NBLK - 1)
    def _():
        out_ref[...] = out_ref[...] * (1.0 / BATCH)


_tc_loss = pl.pallas_call(
    _tc_loss_body,
    grid=(NBLK,),
    in_specs=[
        pl.BlockSpec((1, BLK), lambda i: (0, i)),
        pl.BlockSpec((BLK, 128), lambda i: (i, 0)),
        pl.BlockSpec((1, BLK), lambda i: (0, i)),
        pl.BlockSpec((NUM_SAMPLED, 128), lambda i: (0, 0)),
        pl.BlockSpec((1, NUM_SAMPLED), lambda i: (0, 0)),
        pl.BlockSpec((1, NUM_SAMPLED), lambda i: (0, 0)),
    ],
    out_specs=pl.BlockSpec((1, 1), lambda i: (0, 0)),
    out_shape=jax.ShapeDtypeStruct((1, 1), jnp.float32),
)


def kernel(inputs, labels, embedding, nce_weights, nce_biases, sampled):
    labels_flat = labels.reshape(-1)
    tab = _prep(embedding.T, nce_weights.T)
    p, tb, s, sb = _make_sc_gather()(inputs, labels_flat, tab,
                                     nce_biases, sampled)
    out = _tc_loss(labels_flat.reshape(1, BATCH), p, tb.reshape(1, BATCH),
                   s, sb.reshape(1, NUM_SAMPLED), sampled.reshape(1, NUM_SAMPLED))
    return out[0, 0]


# speedup vs baseline: 1.0170x; 1.0170x over previous
import functools
import math

import jax
import jax.numpy as jnp
from jax import lax
from jax.experimental import pallas as pl
from jax.experimental.pallas import tpu as pltpu
from jax.experimental.pallas import tpu_sc as plsc

VOCAB = 100000
DIM = 64
BATCH = 16384
NUM_SAMPLED = 64

NC, NS = 2, 16
NW = NC * NS
BPW = BATCH // NW
CHUNK = 128
NCHUNK = BPW // CHUNK

_LOG_V1 = math.log(float(VOCAB) + 1.0)


VB = 8192
NVB = -(-VOCAB // VB)


def _prep_body(eT_ref, wT_ref, out_ref):
    out_ref[...] = jnp.concatenate([eT_ref[...].T, wT_ref[...].T], axis=1)


_prep = pl.pallas_call(
    _prep_body,
    grid=(NVB,),
    in_specs=[
        pl.BlockSpec((DIM, VB), lambda i: (0, i)),
        pl.BlockSpec((DIM, VB), lambda i: (0, i)),
    ],
    out_specs=pl.BlockSpec((VB, 128), lambda i: (i, 0)),
    out_shape=jax.ShapeDtypeStruct((VOCAB, 128), jnp.float32),
    compiler_params=pltpu.CompilerParams(vmem_limit_bytes=100 * 1024 * 1024),
)


def _sc_gather_body(inputs_h, labels_h, tab_h, ncb_h, sampled_h,
                    p_o, tb_o, s_o, sb_o,
                    idx_v, lidx_v, a_v, b_v, tb_v, sidx_v, s_v, sb_v,
                    sems, semt):
    wid = lax.axis_index("s") * NC + lax.axis_index("c")
    base = wid * BPW
    for j in range(NCHUNK):
        pltpu.sync_copy(inputs_h.at[pl.ds(base + j * CHUNK, CHUNK)], idx_v.at[j])
        pltpu.sync_copy(labels_h.at[pl.ds(base + j * CHUNK, CHUNK)], lidx_v.at[j])
    tb_handles = [
        pltpu.async_copy(ncb_h.at[lidx_v.at[j]],
                         tb_v.at[pl.ds(j * CHUNK, CHUNK)], semt)
        for j in range(NCHUNK)
    ]
    handles = [None] * NCHUNK

    def fire(j):
        s = j % 2
        handles[j] = (
            pltpu.async_copy(tab_h.at[idx_v.at[j]], a_v.at[s], sems.at[2 * s]),
            pltpu.async_copy(tab_h.at[lidx_v.at[j]], b_v.at[s], sems.at[2 * s + 1]),
        )

    def drain(j):
        ha, hb = handles[j]
        ha.wait()
        hb.wait()
        sl = pl.ds(base + j * CHUNK, CHUNK)
        s = j % 2
        pltpu.sync_copy(a_v.at[s, :, pl.ds(0, DIM)], p_o.at[sl, pl.ds(0, DIM)])
        pltpu.sync_copy(b_v.at[s, :, pl.ds(DIM, DIM)], p_o.at[sl, pl.ds(DIM, DIM)])

    fire(0)
    for j in range(1, NCHUNK):
        fire(j)
        drain(j - 1)
    drain(NCHUNK - 1)

    for h in tb_handles:
        h.wait()
    pltpu.sync_copy(tb_v, tb_o.at[pl.ds(base, BPW)])

    @pl.when(wid == 0)
    def _():
        pltpu.sync_copy(sampled_h, sidx_v)
        pltpu.async_copy(tab_h.at[sidx_v], s_v, semt).wait()
        pltpu.async_copy(ncb_h.at[sidx_v], sb_v, semt).wait()
        pltpu.sync_copy(s_v, s_o)
        pltpu.sync_copy(sb_v, sb_o)


@functools.cache
def _make_sc_gather():
    return functools.partial(
        pl.kernel,
        out_type=[
            jax.ShapeDtypeStruct((BATCH, 128), jnp.float32),
            jax.ShapeDtypeStruct((BATCH,), jnp.float32),
            jax.ShapeDtypeStruct((NUM_SAMPLED, 128), jnp.float32),
            jax.ShapeDtypeStruct((NUM_SAMPLED,), jnp.float32),
        ],
        mesh=plsc.VectorSubcoreMesh(core_axis_name="c", subcore_axis_name="s",
                                    num_cores=NC, num_subcores=NS),
        compiler_params=pltpu.CompilerParams(use_tc_tiling_on_sc=False),
        scratch_types=[
            pltpu.VMEM((NCHUNK, CHUNK), jnp.int32),
            pltpu.VMEM((NCHUNK, CHUNK), jnp.int32),
            pltpu.VMEM((2, CHUNK, 128), jnp.float32),
            pltpu.VMEM((2, CHUNK, 128), jnp.float32),
            pltpu.VMEM((BPW,), jnp.float32),
            pltpu.VMEM((NUM_SAMPLED,), jnp.int32),
            pltpu.VMEM((NUM_SAMPLED, 128), jnp.float32),
            pltpu.VMEM((NUM_SAMPLED,), jnp.float32),
            pltpu.SemaphoreType.DMA((4,)),
            pltpu.SemaphoreType.DMA,
        ],
    )(_sc_gather_body)


BLK = 2048
NBLK = BATCH // BLK


def _tc_loss_body(labels_ref, p_ref, tb_ref, s_ref, sbt_ref,
                  st_ref, out_ref):
    i = pl.program_id(0)
    emb = p_ref[:, pl.ds(0, DIM)]
    tw = p_ref[:, pl.ds(DIM, DIM)]
    tb = tb_ref[...]
    labf = labels_ref[...].astype(jnp.float32)

    true_q = (jnp.log(labf + 2.0) - jnp.log(labf + 1.0)) / _LOG_V1 * NUM_SAMPLED
    tdot = jnp.sum(emb * tw, axis=1, keepdims=True).T
    tl = tdot + tb - jnp.log(true_q)
    ce_true = jnp.maximum(tl, 0.0) - tl + jnp.log1p(jnp.exp(-jnp.abs(tl)))

    sf = st_ref[...].astype(jnp.float32)
    sq = (jnp.log(sf + 2.0) - jnp.log(sf + 1.0)) / _LOG_V1 * NUM_SAMPLED
    sw = s_ref[:, pl.ds(DIM, DIM)]
    slog = lax.dot_general(emb, sw, (((1,), (1,)), ((), ())),
                           preferred_element_type=jnp.float32)
    sl = slog + sbt_ref[...] - jnp.log(sq)
    ce_samp = jnp.maximum(sl, 0.0) + jnp.log1p(jnp.exp(-jnp.abs(sl)))

    part = jnp.broadcast_to(jnp.sum(ce_samp) + jnp.sum(ce_true), (1, 1))

    @pl.when(i == 0)
    def _():
        out_ref[...] = jnp.zeros((1, 1), jnp.float32)

    out_ref[...] += part

    @pl.when(i == NBLK - 1)
    def _():
        out_ref[...] = out_ref[...] * (1.0 / BATCH)


_tc_loss = pl.pallas_call(
    _tc_loss_body,
    grid=(NBLK,),
    in_specs=[
        pl.BlockSpec((1, BLK), lambda i: (0, i)),
        pl.BlockSpec((BLK, 128), lambda i: (i, 0)),
        pl.BlockSpec((1, BLK), lambda i: (0, i)),
        pl.BlockSpec((NUM_SAMPLED, 128), lambda i: (0, 0)),
        pl.BlockSpec((1, NUM_SAMPLED), lambda i: (0, 0)),
        pl.BlockSpec((1, NUM_SAMPLED), lambda i: (0, 0)),
    ],
    out_specs=pl.BlockSpec((1, 1), lambda i: (0, 0)),
    out_shape=jax.ShapeDtypeStruct((1, 1), jnp.float32),
)


def kernel(inputs, labels, embedding, nce_weights, nce_biases, sampled):
    labels_flat = labels.reshape(-1)
    tab = _prep(embedding.T, nce_weights.T)
    p, tb, s, sb = _make_sc_gather()(inputs, labels_flat, tab,
                                     nce_biases, sampled)
    out = _tc_loss(labels_flat.reshape(1, BATCH), p, tb.reshape(1, BATCH),
                   s, sb.reshape(1, NUM_SAMPLED), sampled.reshape(1, NUM_SAMPLED))
    return out[0, 0]


# speedup vs baseline: 1.0229x; 1.0058x over previous
import functools
import math

import jax
import jax.numpy as jnp
from jax import lax
from jax.experimental import pallas as pl
from jax.experimental.pallas import tpu as pltpu
from jax.experimental.pallas import tpu_sc as plsc

VOCAB = 100000
DIM = 64
BATCH = 16384
NUM_SAMPLED = 64

NC, NS = 2, 16
NW = NC * NS
BPW = BATCH // NW
CHUNK = 128
NCHUNK = BPW // CHUNK

_LOG_V1 = math.log(float(VOCAB) + 1.0)


VB = 8192
NVB = -(-VOCAB // VB)


def _prep_body(eT_ref, wT_ref, out_ref):
    out_ref[...] = jnp.concatenate([eT_ref[...].T, wT_ref[...].T], axis=1)


_prep = pl.pallas_call(
    _prep_body,
    grid=(NVB,),
    in_specs=[
        pl.BlockSpec((DIM, VB), lambda i: (0, i)),
        pl.BlockSpec((DIM, VB), lambda i: (0, i)),
    ],
    out_specs=pl.BlockSpec((VB, 128), lambda i: (i, 0)),
    out_shape=jax.ShapeDtypeStruct((VOCAB, 128), jnp.float32),
    compiler_params=pltpu.CompilerParams(vmem_limit_bytes=100 * 1024 * 1024),
)


def _sc_gather_body(inputs_h, labels_h, tab_h, ncb_h, sampled_h,
                    p_o, tb_o, s_o, sb_o,
                    idx_v, lidx_v, a_v, b_v, tb_v, sidx_v, s_v, sb_v,
                    sems, semt):
    wid = lax.axis_index("s") * NC + lax.axis_index("c")
    base = wid * BPW
    for j in range(NCHUNK):
        pltpu.sync_copy(inputs_h.at[pl.ds(base + j * CHUNK, CHUNK)], idx_v.at[j])
        pltpu.sync_copy(labels_h.at[pl.ds(base + j * CHUNK, CHUNK)], lidx_v.at[j])
    tb_handles = [
        pltpu.async_copy(ncb_h.at[lidx_v.at[j]],
                         tb_v.at[pl.ds(j * CHUNK, CHUNK)], semt)
        for j in range(NCHUNK)
    ]
    handles = [None] * NCHUNK

    def fire(j):
        s = j % 2
        handles[j] = (
            pltpu.async_copy(tab_h.at[idx_v.at[j]], a_v.at[s], sems.at[2 * s]),
            pltpu.async_copy(tab_h.at[lidx_v.at[j]], b_v.at[s], sems.at[2 * s + 1]),
        )

    def drain(j):
        ha, hb = handles[j]
        ha.wait()
        hb.wait()
        sl = pl.ds(base + j * CHUNK, CHUNK)
        s = j % 2
        pltpu.sync_copy(a_v.at[s, :, pl.ds(0, DIM)], p_o.at[sl, pl.ds(0, DIM)])
        pltpu.sync_copy(b_v.at[s, :, pl.ds(DIM, DIM)], p_o.at[sl, pl.ds(DIM, DIM)])

    fire(0)
    for j in range(1, NCHUNK):
        fire(j)
        drain(j - 1)
    drain(NCHUNK - 1)

    for h in tb_handles:
        h.wait()
    pltpu.sync_copy(tb_v, tb_o.at[pl.ds(base, BPW)])

    @pl.when(wid == 0)
    def _():
        pltpu.sync_copy(sampled_h, sidx_v)
        pltpu.async_copy(tab_h.at[sidx_v], s_v, semt).wait()
        pltpu.async_copy(ncb_h.at[sidx_v], sb_v, semt).wait()
        pltpu.sync_copy(s_v, s_o)
        pltpu.sync_copy(sb_v, sb_o)


@functools.cache
def _make_sc_gather():
    return functools.partial(
        pl.kernel,
        out_type=[
            jax.ShapeDtypeStruct((BATCH, 128), jnp.float32),
            jax.ShapeDtypeStruct((BATCH,), jnp.float32),
            jax.ShapeDtypeStruct((NUM_SAMPLED, 128), jnp.float32),
            jax.ShapeDtypeStruct((NUM_SAMPLED,), jnp.float32),
        ],
        mesh=plsc.VectorSubcoreMesh(core_axis_name="c", subcore_axis_name="s",
                                    num_cores=NC, num_subcores=NS),
        compiler_params=pltpu.CompilerParams(use_tc_tiling_on_sc=False),
        scratch_types=[
            pltpu.VMEM((NCHUNK, CHUNK), jnp.int32),
            pltpu.VMEM((NCHUNK, CHUNK), jnp.int32),
            pltpu.VMEM((2, CHUNK, 128), jnp.float32),
            pltpu.VMEM((2, CHUNK, 128), jnp.float32),
            pltpu.VMEM((BPW,), jnp.float32),
            pltpu.VMEM((NUM_SAMPLED,), jnp.int32),
            pltpu.VMEM((NUM_SAMPLED, 128), jnp.float32),
            pltpu.VMEM((NUM_SAMPLED,), jnp.float32),
            pltpu.SemaphoreType.DMA((4,)),
            pltpu.SemaphoreType.DMA,
        ],
    )(_sc_gather_body)


BLK = 4096
NBLK = BATCH // BLK


def _tc_loss_body(labels_ref, p_ref, tb_ref, s_ref, sbt_ref,
                  st_ref, out_ref):
    i = pl.program_id(0)
    emb = p_ref[:, pl.ds(0, DIM)]
    tw = p_ref[:, pl.ds(DIM, DIM)]
    tb = tb_ref[...]
    labf = labels_ref[...].astype(jnp.float32)

    true_q = (jnp.log(labf + 2.0) - jnp.log(labf + 1.0)) / _LOG_V1 * NUM_SAMPLED
    tdot = jnp.sum(emb * tw, axis=1, keepdims=True).T
    tl = tdot + tb - jnp.log(true_q)
    ce_true = jnp.maximum(tl, 0.0) - tl + jnp.log1p(jnp.exp(-jnp.abs(tl)))

    sf = st_ref[...].astype(jnp.float32)
    sq = (jnp.log(sf + 2.0) - jnp.log(sf + 1.0)) / _LOG_V1 * NUM_SAMPLED
    sw = s_ref[:, pl.ds(DIM, DIM)]
    slog = lax.dot_general(emb, sw, (((1,), (1,)), ((), ())),
                           preferred_element_type=jnp.float32)
    sl = slog + sbt_ref[...] - jnp.log(sq)
    ce_samp = jnp.maximum(sl, 0.0) + jnp.log1p(jnp.exp(-jnp.abs(sl)))

    part = jnp.broadcast_to(jnp.sum(ce_samp) + jnp.sum(ce_true), (1, 1))

    @pl.when(i == 0)
    def _():
        out_ref[...] = jnp.zeros((1, 1), jnp.float32)

    out_ref[...] += part

    @pl.when(i == NBLK - 1)
    def _():
        out_ref[...] = out_ref[...] * (1.0 / BATCH)


_tc_loss = pl.pallas_call(
    _tc_loss_body,
    grid=(NBLK,),
    in_specs=[
        pl.BlockSpec((1, BLK), lambda i: (0, i)),
        pl.BlockSpec((BLK, 128), lambda i: (i, 0)),
        pl.BlockSpec((1, BLK), lambda i: (0, i)),
        pl.BlockSpec((NUM_SAMPLED, 128), lambda i: (0, 0)),
        pl.BlockSpec((1, NUM_SAMPLED), lambda i: (0, 0)),
        pl.BlockSpec((1, NUM_SAMPLED), lambda i: (0, 0)),
    ],
    out_specs=pl.BlockSpec((1, 1), lambda i: (0, 0)),
    out_shape=jax.ShapeDtypeStruct((1, 1), jnp.float32),
)


def kernel(inputs, labels, embedding, nce_weights, nce_biases, sampled):
    labels_flat = labels.reshape(-1)
    tab = _prep(embedding.T, nce_weights.T)
    p, tb, s, sb = _make_sc_gather()(inputs, labels_flat, tab,
                                     nce_biases, sampled)
    out = _tc_loss(labels_flat.reshape(1, BATCH), p, tb.reshape(1, BATCH),
                   s, sb.reshape(1, NUM_SAMPLED), sampled.reshape(1, NUM_SAMPLED))
    return out[0, 0]
